# baseline (speedup 1.0000x reference)
import jax
import jax.numpy as jnp
import numpy as np
from jax.experimental import pallas as pl
from jax.experimental.pallas import tpu as pltpu
from jax.sharding import Mesh, PartitionSpec as P

try:
    from jax.experimental.shard_map import shard_map as _shard_map_raw
except ImportError:
    _shard_map_raw = jax.shard_map


def _shard_map(f, **kw):
    try:
        return _shard_map_raw(f, check_vma=False, **kw)
    except TypeError:
        return _shard_map_raw(f, check_rep=False, **kw)

_TILE_M = 1024


def _ffn_body(x_ref, w1_ref, b1_ref, w2_ref, b2_ref, o_ref):
    xb = x_ref[...].astype(jnp.bfloat16)
    h = jnp.dot(xb, w1_ref[...], preferred_element_type=jnp.float32)
    h = jnp.maximum(h + b1_ref[...], 0.0).astype(jnp.bfloat16)
    y = jnp.dot(h, w2_ref[...], preferred_element_type=jnp.float32)
    o_ref[...] = (y + b2_ref[...]).astype(o_ref.dtype)


def _ffn_call(m_rows, tile_m, d_in, d_mid, d_out, out_dtype):
    const = lambda i: (0, 0)
    wkw = {"pipeline_mode": pl.Buffered(1)}
    return pl.pallas_call(
        _ffn_body,
        out_shape=jax.ShapeDtypeStruct((m_rows, d_out), out_dtype),
        grid=(m_rows // tile_m,),
        in_specs=[
            pl.BlockSpec((tile_m, d_in), lambda i: (i, 0)),
            pl.BlockSpec((d_in, d_mid), const, **wkw),
            pl.BlockSpec((1, d_mid), const, **wkw),
            pl.BlockSpec((d_mid, d_out), const, **wkw),
            pl.BlockSpec((1, d_out), const, **wkw),
        ],
        out_specs=pl.BlockSpec((tile_m, d_out), lambda i: (i, 0)),
        compiler_params=pltpu.CompilerParams(
            dimension_semantics=("arbitrary",),
            vmem_limit_bytes=60 * 1024 * 1024,
        ),
    )


def _forward(x2, w1b, b1f, w2b, b2f):
    m_rows, d_in = x2.shape
    d_mid = w1b.shape[1]
    tile_m = min(_TILE_M, m_rows)
    while m_rows % tile_m:
        tile_m //= 2
    return _ffn_call(m_rows, tile_m, d_in, d_mid, d_in, x2.dtype)(
        x2, w1b, b1f, w2b, b2f)


@jax.jit
def kernel(x, w1, b1, w2, b2):
    B, S, H = x.shape
    FF = w1.shape[1]
    M = B * S
    x2 = x.reshape(M, H)

    w1b = w1.astype(jnp.bfloat16)
    w2b = w2.astype(jnp.bfloat16)
    b1f = b1.astype(jnp.float32).reshape(1, FF)
    b2f = b2.astype(jnp.float32).reshape(1, H)

    devs = jax.devices()[:2]
    if len(devs) == 2 and M % (2 * 8) == 0:
        mesh = Mesh(np.asarray(devs), ("d",))
        fwd = _shard_map(
            _forward, mesh=mesh,
            in_specs=(P("d", None), P(None, None), P(None, None),
                      P(None, None), P(None, None)),
            out_specs=P("d", None),
        )
    else:
        fwd = _forward
    out = fwd(x2, w1b, b1f, w2b, b2f)
    return out.reshape(B, S, H)


# speedup vs baseline: 3.0832x; 3.0832x over previous
import jax
import jax.numpy as jnp
from jax.experimental import pallas as pl
from jax.experimental.pallas import tpu as pltpu

_TILE_M = 1024


def _ffn_body(x_ref, w1_ref, b1_ref, w2_ref, b2_ref, o_ref, w1b_ref, w2b_ref):
    @pl.when(pl.program_id(0) == 0)
    def _cast_weights():
        w1b_ref[...] = w1_ref[...].astype(jnp.bfloat16)
        w2b_ref[...] = w2_ref[...].astype(jnp.bfloat16)

    xb = x_ref[...].astype(jnp.bfloat16)
    h = jnp.dot(xb, w1b_ref[...], preferred_element_type=jnp.float32)
    h = jnp.maximum(h + b1_ref[...], 0.0).astype(jnp.bfloat16)
    y = jnp.dot(h, w2b_ref[...], preferred_element_type=jnp.float32)
    o_ref[...] = (y + b2_ref[...]).astype(o_ref.dtype)


def _ffn_call(m_rows, tile_m, d_in, d_mid, d_out, out_dtype):
    const = lambda i: (0, 0)
    wkw = {"pipeline_mode": pl.Buffered(1)}
    return pl.pallas_call(
        _ffn_body,
        out_shape=jax.ShapeDtypeStruct((m_rows, d_out), out_dtype),
        grid=(m_rows // tile_m,),
        in_specs=[
            pl.BlockSpec((tile_m, d_in), lambda i: (i, 0)),
            pl.BlockSpec((d_in, d_mid), const, **wkw),
            pl.BlockSpec((1, d_mid), const, **wkw),
            pl.BlockSpec((d_mid, d_out), const, **wkw),
            pl.BlockSpec((1, d_out), const, **wkw),
        ],
        out_specs=pl.BlockSpec((tile_m, d_out), lambda i: (i, 0)),
        scratch_shapes=[
            pltpu.VMEM((d_in, d_mid), jnp.bfloat16),
            pltpu.VMEM((d_mid, d_out), jnp.bfloat16),
        ],
        compiler_params=pltpu.CompilerParams(
            dimension_semantics=("arbitrary",),
            vmem_limit_bytes=60 * 1024 * 1024,
        ),
    )


@jax.jit
def kernel(x, w1, b1, w2, b2):
    B, S, H = x.shape
    FF = w1.shape[1]
    M = B * S
    x2 = x.reshape(M, H)

    b1f = b1.astype(jnp.float32).reshape(1, FF)
    b2f = b2.astype(jnp.float32).reshape(1, H)

    tile_m = min(_TILE_M, M)
    while M % tile_m:
        tile_m //= 2

    out = _ffn_call(M, tile_m, H, FF, H, x.dtype)(x2, w1, b1f, w2, b2f)
    return out.reshape(B, S, H)
